# Initial kernel scaffold; baseline (speedup 1.0000x reference)
import jax
import jax.numpy as jnp
from jax.experimental import pallas as pl


def kernel(pos, batch, W1, b1, g1, be1, W2, b2, g2, be2, W3, b3, g3, be3, W4, b4, g4, be4, Wc1, bc1, Wc2, bc2):
    raise NotImplementedError("write your pallas kernel here")


# speedup vs baseline: 9.1923x; 9.1923x over previous
import functools

import jax
import jax.numpy as jnp
from jax import lax
from jax.experimental import pallas as pl
from jax.experimental.pallas import tpu as pltpu
from jax.experimental.pallas import tpu_sc as plsc

_B, _P, _K = 16, 1024, 8
_N = _B * _P
_E = _N * _K
_D = 128
_ROWS = 256
_EP = 512

_NC, _NS = 2, 16
_NW = _NC * _NS
_EW = _E // _NW
_ECH = 128


def _knn_body(xblk_ref, xall_ref, idx_ref):
    b = pl.program_id(0)
    t = pl.program_id(1)
    rows = xblk_ref[0]
    x_all = xall_ref[0]

    g = lax.dot_general(rows, x_all, (((1,), (1,)), ((), ())),
                        preferred_element_type=jnp.float32)
    sq_all = jnp.sum(x_all * x_all, axis=1)
    sq_rows = jnp.sum(rows * rows, axis=1, keepdims=True)
    s = (sq_rows + sq_all[None, :]) - 2.0 * g

    col_ids = lax.broadcasted_iota(jnp.int32, (_ROWS, _P), 1)
    row_ids = t * _ROWS + lax.broadcasted_iota(jnp.int32, (_ROWS, 1), 0)
    s = jnp.where(col_ids == row_ids, s + 1e10, s)

    kio = lax.broadcasted_iota(jnp.int32, (_ROWS, _K), 1)
    idxm = jnp.zeros((_ROWS, _K), jnp.int32)
    big = jnp.float32(3.0e38)
    for k in range(_K):
        m = jnp.min(s, axis=1, keepdims=True)
        am = jnp.min(jnp.where(s == m, col_ids, _P), axis=1, keepdims=True)
        idxm = jnp.where(kio == k, am, idxm)
        if k < _K - 1:
            s = jnp.where(col_ids == am, big, s)
    idx_ref[0] = idxm + b * _P


def _knn(x3):
    return pl.pallas_call(
        _knn_body,
        grid=(_B, _P // _ROWS),
        in_specs=[
            pl.BlockSpec((1, _ROWS, _D), lambda b, t: (b, t, 0)),
            pl.BlockSpec((1, _P, _D), lambda b, t: (b, 0, 0)),
        ],
        out_specs=pl.BlockSpec((1, _ROWS, _K), lambda b, t: (b, t, 0)),
        out_shape=jax.ShapeDtypeStruct((_B, _P, _K), jnp.int32),
    )(x3, x3)


def _make_gather():
    mesh = plsc.VectorSubcoreMesh(core_axis_name="c", subcore_axis_name="s")
    n_chunks = _EW // _ECH

    @functools.partial(
        pl.kernel,
        mesh=mesh,
        out_type=jax.ShapeDtypeStruct((_E, _D), jnp.float32),
        scratch_types=[
            pltpu.VMEM((_ECH,), jnp.int32),
            pltpu.VMEM((_ECH, _D), jnp.float32),
            pltpu.SemaphoreType.DMA,
        ],
    )
    def gather(x_hbm, idx_hbm, xj_hbm, idx_v, rows_v, sem):
        wid = lax.axis_index("s") * _NC + lax.axis_index("c")
        base = wid * _EW

        def chunk(c, carry):
            eb = base + c * _ECH
            pltpu.sync_copy(idx_hbm.at[pl.ds(eb, _ECH)], idx_v)
            pltpu.async_copy(x_hbm.at[idx_v], rows_v, sem).wait()
            pltpu.sync_copy(rows_v, xj_hbm.at[pl.ds(eb, _ECH)])
            return carry

        lax.fori_loop(0, n_chunks, chunk, 0)

    return gather


_GATHER = None


def _gather_rows(x_pad, idx_e):
    global _GATHER
    if _GATHER is None:
        _GATHER = _make_gather()
    return _GATHER(x_pad, idx_e)


def _edge_h(xi, xj, w_ref, b_ref):
    m = jnp.concatenate([xi - xj, xj], axis=1)
    return jnp.dot(m, w_ref[...],
                   preferred_element_type=jnp.float32) + b_ref[...]


def _stats_body(xj_ref, xi_ref, w_ref, b_ref, out_ref):
    k = pl.program_id(0)
    t = pl.program_id(1)
    h = _edge_h(xi_ref[...], xj_ref[0], w_ref, b_ref)
    r0 = jnp.sum(h, axis=0)
    r1 = jnp.sum(h * h, axis=0)
    rio = lax.broadcasted_iota(jnp.int32, (16, h.shape[-1]), 0)
    part = (jnp.where(rio == k, r0[None, :], 0.0)
            + jnp.where(rio == k + 8, r1[None, :], 0.0))

    @pl.when(jnp.logical_and(k == 0, t == 0))
    def _():
        out_ref[...] = jnp.zeros_like(out_ref)

    out_ref[...] += part


def _bn_stats(xj3, x_pad, w, bvec):
    dout = w.shape[-1]
    return pl.pallas_call(
        _stats_body,
        grid=(_K, _N // _EP),
        in_specs=[
            pl.BlockSpec((1, _EP, _D), lambda k, t: (k, t, 0)),
            pl.BlockSpec((_EP, _D), lambda k, t: (t, 0)),
            pl.BlockSpec((2 * _D, dout), lambda k, t: (0, 0)),
            pl.BlockSpec((1, dout), lambda k, t: (0, 0)),
        ],
        out_specs=pl.BlockSpec((16, dout), lambda k, t: (0, 0)),
        out_shape=jax.ShapeDtypeStruct((16, dout), jnp.float32),
    )(xj3, x_pad, w, bvec)


def _out_body(xj_ref, xi_ref, w_ref, b_ref, st_ref, g_ref, be_ref, out_ref,
              *, dpad):
    xi = xi_ref[...]
    hmax = _edge_h(xi, xj_ref[0], w_ref, b_ref)
    for k in range(1, _K):
        hmax = jnp.maximum(hmax, _edge_h(xi, xj_ref[k], w_ref, b_ref))
    st = st_ref[...]
    inv_e = jnp.float32(1.0 / _E)
    s1 = ((st[0] + st[1]) + (st[2] + st[3])) \
        + ((st[4] + st[5]) + (st[6] + st[7]))
    s2 = ((st[8] + st[9]) + (st[10] + st[11])) \
        + ((st[12] + st[13]) + (st[14] + st[15]))
    mu = s1 * inv_e
    var = s2 * inv_e - mu * mu
    t = g_ref[...] * (hmax - mu[None, :])
    t = t / jnp.sqrt(var + 1e-5)[None, :]
    o = jnp.maximum(t + be_ref[...], 0.0)
    if dpad:
        o = jnp.pad(o, ((0, 0), (0, dpad)))
    out_ref[...] = o


def _edge_out(xj3, x_pad, w, bvec, st, g, be):
    dout = w.shape[-1]
    dpad = max(_D - dout, 0)
    return pl.pallas_call(
        functools.partial(_out_body, dpad=dpad),
        grid=(_N // _EP,),
        in_specs=[
            pl.BlockSpec((_K, _EP, _D), lambda t: (0, t, 0)),
            pl.BlockSpec((_EP, _D), lambda t: (t, 0)),
            pl.BlockSpec((2 * _D, dout), lambda t: (0, 0)),
            pl.BlockSpec((1, dout), lambda t: (0, 0)),
            pl.BlockSpec((16, dout), lambda t: (0, 0)),
            pl.BlockSpec((1, dout), lambda t: (0, 0)),
            pl.BlockSpec((1, dout), lambda t: (0, 0)),
        ],
        out_specs=pl.BlockSpec((_EP, dout + dpad), lambda t: (t, 0)),
        out_shape=jax.ShapeDtypeStruct((_N, dout + dpad), jnp.float32),
    )(xj3, x_pad, w, bvec, st, g, be)


def _pool_body(x_ref, out_ref):
    out_ref[0, 0] = jnp.max(x_ref[0], axis=0)


def _pool(x3):
    d = x3.shape[-1]
    return pl.pallas_call(
        _pool_body,
        grid=(_B,),
        in_specs=[pl.BlockSpec((1, _P, d), lambda b: (b, 0, 0))],
        out_specs=pl.BlockSpec((1, 1, d), lambda b: (b, 0, 0)),
        out_shape=jax.ShapeDtypeStruct((_B, 1, d), jnp.float32),
    )(x3).reshape(_B, d)


def _head_body(p_ref, w1_ref, b1_ref, w2_ref, b2_ref, out_ref):
    h = jnp.dot(p_ref[...], w1_ref[...],
                preferred_element_type=jnp.float32) + b1_ref[...]
    h = jnp.maximum(h, 0.0)
    out_ref[...] = jnp.dot(h, w2_ref[...],
                           preferred_element_type=jnp.float32) + b2_ref[...]


def _head(pooled, wc1, bc1, wc2, bc2):
    return pl.pallas_call(
        _head_body,
        out_shape=jax.ShapeDtypeStruct((_B, 40), jnp.float32),
    )(pooled, wc1, bc1.reshape(1, -1), wc2, bc2.reshape(1, -1))


def _edge_layer(x_pad, w, bvec, g, be):
    din = w.shape[0] // 2
    dout = w.shape[-1]
    wp = jnp.zeros((2 * _D, dout), jnp.float32)
    wp = wp.at[:din].set(w[:din]).at[_D:_D + din].set(w[din:])
    idx = _knn(x_pad.reshape(_B, _P, _D))
    idx_e = idx.reshape(_N, _K).transpose(1, 0).reshape(_E)
    xj = _gather_rows(x_pad, idx_e)
    xj3 = xj.reshape(_K, _N, _D)
    st = _bn_stats(xj3, x_pad, wp, bvec.reshape(1, dout))
    return _edge_out(xj3, x_pad, wp, bvec.reshape(1, dout), st,
                     g.reshape(1, dout), be.reshape(1, dout))


def kernel(pos, batch, W1, b1, g1, be1, W2, b2, g2, be2, W3, b3, g3, be3,
           W4, b4, g4, be4, Wc1, bc1, Wc2, bc2):
    x = jnp.pad(pos, ((0, 0), (0, _D - pos.shape[-1])))
    x = _edge_layer(x, W1, b1, g1, be1)
    x = _edge_layer(x, W2, b2, g2, be2)
    x = _edge_layer(x, W3, b3, g3, be3)
    x = _edge_layer(x, W4, b4, g4, be4)
    pooled = _pool(x.reshape(_B, _P, x.shape[-1]))
    return _head(pooled, Wc1, bc1, Wc2, bc2)
